# baseline (speedup 1.0000x reference)
import jax
import jax.numpy as jnp
from jax.experimental import pallas as pl
from jax.experimental.pallas import tpu as pltpu

N = 10000
D = 128
BM = 200
STEPS = N // (2 * BM)


def _gcn_body(adj0_ref, adj1_ref, emb_ref, out_ref, emb_bf_ref):
    @pl.when(pl.program_id(0) == 0)
    def _():
        emb_bf_ref[...] = emb_ref[...].astype(jnp.bfloat16)

    e = emb_bf_ref[...]
    out_ref[0:BM, :] = jnp.dot(
        adj0_ref[...].astype(jnp.bfloat16), e, preferred_element_type=jnp.float32)
    out_ref[BM:2 * BM, :] = jnp.dot(
        adj1_ref[...].astype(jnp.bfloat16), e, preferred_element_type=jnp.float32)


def kernel(adj, embeds):
    return pl.pallas_call(
        _gcn_body,
        grid=(STEPS,),
        in_specs=[
            pl.BlockSpec((BM, N), lambda i: (2 * i, 0)),
            pl.BlockSpec((BM, N), lambda i: (2 * i + 1, 0)),
            pl.BlockSpec((N, D), lambda i: (0, 0)),
        ],
        out_specs=pl.BlockSpec((2 * BM, D), lambda i: (i, 0)),
        out_shape=jax.ShapeDtypeStruct((N, D), jnp.float32),
        scratch_shapes=[pltpu.VMEM((N, D), jnp.bfloat16)],
        compiler_params=pltpu.CompilerParams(
            dimension_semantics=("arbitrary",),
        ),
    )(adj, adj, embeds)
